# Initial kernel scaffold; baseline (speedup 1.0000x reference)
import jax
import jax.numpy as jnp
from jax.experimental import pallas as pl


def kernel(tensor, indices, Uf_w, Uf_b, Uiuo_w, Uiuo_b, W_w, W_b):
    raise NotImplementedError("write your pallas kernel here")


# speedup vs baseline: 1.1165x; 1.1165x over previous
import functools

import jax
import jax.numpy as jnp
from jax import lax
from jax.experimental import pallas as pl
from jax.experimental.pallas import tpu as pltpu
from jax.experimental.pallas import tpu_sc as plsc

L, N, K, DIN, DOUT = 8, 12500, 4, 128, 128
DHC = 2 * DOUT
NP = 12800
NB = 640
NBLK = NP // NB
NW = 32
CH = 128
NCH = 13
PERW = NCH * CH
BG = NW * PERW


def _gather_body(table_hbm, idx_hbm, out_hbm, idx_v, buf_v, sem):
    wid = lax.axis_index("s") * 2 + lax.axis_index("c")
    pltpu.sync_copy(idx_hbm.at[wid], idx_v)
    base = wid * PERW
    for j in range(NCH):
        pltpu.async_copy(table_hbm.at[idx_v.at[j]], buf_v.at[j % 2], sem).wait()
        pltpu.sync_copy(buf_v.at[j % 2], out_hbm.at[pl.ds(base + j * CH, CH)])


@functools.cache
def _gather_call():
    return pl.kernel(
        _gather_body,
        mesh=plsc.VectorSubcoreMesh(core_axis_name="c", subcore_axis_name="s",
                                    num_cores=2),
        out_type=jax.ShapeDtypeStruct((BG, DHC), jnp.float32),
        scratch_types=[
            pltpu.VMEM((NCH, CH), jnp.int32),
            pltpu.VMEM((2, CH, DHC), jnp.float32),
            pltpu.SemaphoreType.DMA,
        ],
    )


def _sigmoid(x):
    return 1.0 / (1.0 + jnp.exp(-x))


def _lvl0_body(x_ref, Ww_ref, Wb_ref, Ub_ref, h_ref, c_ref):
    wx = jnp.dot(x_ref[...], Ww_ref[...],
                 preferred_element_type=jnp.float32) + Wb_ref[...]
    ub = Ub_ref[...]
    i = _sigmoid(ub[:, :DOUT] + wx[:, DOUT:2 * DOUT])
    u = jnp.tanh(ub[:, DOUT:2 * DOUT] + wx[:, 2 * DOUT:3 * DOUT])
    o = _sigmoid(ub[:, 2 * DOUT:] + wx[:, 3 * DOUT:])
    nc = i * u
    c_ref[...] = nc
    h_ref[...] = o * jnp.tanh(nc)


def _lvl_body(g_ref, x_ref, Ww_ref, Wb_ref, Ufw_ref, Ufb_ref, Uw_ref, Ub_ref,
              h_ref, c_ref):
    wx = jnp.dot(x_ref[...], Ww_ref[...],
                 preferred_element_type=jnp.float32) + Wb_ref[...]
    g = g_ref[...]
    wfx = wx[:, :DOUT]
    ufb = Ufb_ref[...]
    h_sum = jnp.zeros((NB, DOUT), jnp.float32)
    bf = jnp.zeros((NB, DOUT), jnp.float32)
    for k in range(K):
        hk = g[:, k * DHC:k * DHC + DOUT]
        ck = g[:, k * DHC + DOUT:(k + 1) * DHC]
        h_sum = h_sum + hk
        fUk = jnp.dot(hk, Ufw_ref[...], preferred_element_type=jnp.float32)
        bf = bf + _sigmoid(wfx + fUk + ufb) * ck
    iuo = jnp.dot(h_sum, Uw_ref[...],
                  preferred_element_type=jnp.float32) + Ub_ref[...]
    i = _sigmoid(iuo[:, :DOUT] + wx[:, DOUT:2 * DOUT])
    u = jnp.tanh(iuo[:, DOUT:2 * DOUT] + wx[:, 2 * DOUT:3 * DOUT])
    o = _sigmoid(iuo[:, 2 * DOUT:] + wx[:, 3 * DOUT:])
    nc = i * u + bf
    c_ref[...] = nc
    h_ref[...] = o * jnp.tanh(nc)


def _full(shape):
    return pl.BlockSpec(shape, lambda j: (0, 0))


def _lvl0_call(x, Ww, Wb2, Ub2):
    return pl.pallas_call(
        _lvl0_body,
        grid=(NBLK,),
        in_specs=[
            pl.BlockSpec((NB, DIN), lambda j: (j, 0)),
            _full((DIN, 4 * DOUT)),
            _full((1, 4 * DOUT)),
            _full((1, 3 * DOUT)),
        ],
        out_specs=[pl.BlockSpec((NB, DOUT), lambda j: (j, 0))] * 2,
        out_shape=[jax.ShapeDtypeStruct((NP, DOUT), jnp.float32)] * 2,
    )(x, Ww, Wb2, Ub2)


def _lvl_call(g, x, Ww, Wb2, Ufw, Ufb2, Uw, Ub2):
    return pl.pallas_call(
        _lvl_body,
        grid=(NBLK,),
        in_specs=[
            pl.BlockSpec((NB, K * DHC), lambda j: (j, 0)),
            pl.BlockSpec((NB, DIN), lambda j: (j, 0)),
            _full((DIN, 4 * DOUT)),
            _full((1, 4 * DOUT)),
            _full((DOUT, DOUT)),
            _full((1, DOUT)),
            _full((DOUT, 3 * DOUT)),
            _full((1, 3 * DOUT)),
        ],
        out_specs=[pl.BlockSpec((NB, DOUT), lambda j: (j, 0))] * 2,
        out_shape=[jax.ShapeDtypeStruct((NP, DOUT), jnp.float32)] * 2,
    )(g, x, Ww, Wb2, Ufw, Ufb2, Uw, Ub2)


def kernel(tensor, indices, Uf_w, Uf_b, Uiuo_w, Uiuo_b, W_w, W_b):
    xpad = jnp.pad(tensor, ((0, 0), (0, NP - N), (0, 0)))
    Wb2 = W_b.reshape(1, 4 * DOUT)
    Ufb2 = Uf_b.reshape(1, DOUT)
    Ub2 = Uiuo_b.reshape(1, 3 * DOUT)

    h_prev, c_prev = _lvl0_call(xpad[0], W_w, Wb2, Ub2)
    res_h, res_c = [h_prev[:N]], [c_prev[:N]]
    for l in range(1, L):
        table = jnp.concatenate(
            [jnp.zeros((1, DHC), jnp.float32),
             jnp.concatenate([h_prev[:N], c_prev[:N]], axis=1)], axis=0)
        idx = jnp.maximum(indices[l], 0).reshape(-1)
        idx = jnp.pad(idx, (0, BG - N * K)).reshape(NW, NCH, CH)
        g = _gather_call()(table, idx)
        g = g[:NP * K].reshape(NP, K * DHC)
        h_prev, c_prev = _lvl_call(g, xpad[l], W_w, Wb2, Uf_w, Ufb2,
                                   Uiuo_w, Ub2)
        res_h.append(h_prev[:N])
        res_c.append(c_prev[:N])
    return jnp.stack(res_h), jnp.stack(res_c)


# speedup vs baseline: 1.1457x; 1.0261x over previous
import functools

import jax
import jax.numpy as jnp
from jax import lax
from jax.experimental import pallas as pl
from jax.experimental.pallas import tpu as pltpu
from jax.experimental.pallas import tpu_sc as plsc

L, N, K, DIN, DOUT = 8, 12500, 4, 128, 128
DHC = 2 * DOUT
NP = 12800
NB = 640
NBLK = NP // NB
NW = 32
CH = 128
NCH = 13
PERW = NCH * CH
BG = NW * PERW


NBUF = 3


def _gather_body(table_hbm, idx_hbm, out_hbm, idx_v, buf_v,
                 g0, g1, g2, w0, w1, w2):
    gsem = [g0, g1, g2]
    wsem = [w0, w1, w2]
    wid = lax.axis_index("s") * 2 + lax.axis_index("c")
    pltpu.sync_copy(idx_hbm.at[wid], idx_v)
    base = wid * PERW
    gt = [None] * NBUF
    wb = [None] * NBUF
    for b in range(NBUF):
        gt[b] = pltpu.async_copy(table_hbm.at[idx_v.at[b]], buf_v.at[b],
                                 gsem[b])
    for j in range(NCH):
        b = j % NBUF
        gt[b].wait()
        wb[b] = pltpu.async_copy(buf_v.at[b],
                                 out_hbm.at[pl.ds(base + j * CH, CH)],
                                 wsem[b])
        n = j + NBUF
        if n < NCH:
            wb[b].wait()
            gt[b] = pltpu.async_copy(table_hbm.at[idx_v.at[n]], buf_v.at[b],
                                     gsem[b])
    for j in range(NCH - NBUF, NCH):
        wb[j % NBUF].wait()


@functools.cache
def _gather_call():
    return pl.kernel(
        _gather_body,
        mesh=plsc.VectorSubcoreMesh(core_axis_name="c", subcore_axis_name="s",
                                    num_cores=2),
        out_type=jax.ShapeDtypeStruct((BG, DHC), jnp.float32),
        scratch_types=[
            pltpu.VMEM((NCH, CH), jnp.int32),
            pltpu.VMEM((NBUF, CH, DHC), jnp.float32),
        ] + [pltpu.SemaphoreType.DMA] * (2 * NBUF),
    )


def _sigmoid(x):
    return 1.0 / (1.0 + jnp.exp(-x))


def _lvl0_body(x_ref, Ww_ref, Wb_ref, Ub_ref, h_ref, c_ref):
    wx = jnp.dot(x_ref[...], Ww_ref[...],
                 preferred_element_type=jnp.float32) + Wb_ref[...]
    ub = Ub_ref[...]
    i = _sigmoid(ub[:, :DOUT] + wx[:, DOUT:2 * DOUT])
    u = jnp.tanh(ub[:, DOUT:2 * DOUT] + wx[:, 2 * DOUT:3 * DOUT])
    o = _sigmoid(ub[:, 2 * DOUT:] + wx[:, 3 * DOUT:])
    nc = i * u
    c_ref[...] = nc
    h_ref[...] = o * jnp.tanh(nc)


def _lvl_body(g_ref, x_ref, Ww_ref, Wb_ref, Ufw_ref, Ufb_ref, Uw_ref, Ub_ref,
              h_ref, c_ref):
    wx = jnp.dot(x_ref[...], Ww_ref[...],
                 preferred_element_type=jnp.float32) + Wb_ref[...]
    g = g_ref[...]
    wfx = wx[:, :DOUT]
    ufb = Ufb_ref[...]
    h_sum = jnp.zeros((NB, DOUT), jnp.float32)
    bf = jnp.zeros((NB, DOUT), jnp.float32)
    for k in range(K):
        hk = g[:, k * DHC:k * DHC + DOUT]
        ck = g[:, k * DHC + DOUT:(k + 1) * DHC]
        h_sum = h_sum + hk
        fUk = jnp.dot(hk, Ufw_ref[...], preferred_element_type=jnp.float32)
        bf = bf + _sigmoid(wfx + fUk + ufb) * ck
    iuo = jnp.dot(h_sum, Uw_ref[...],
                  preferred_element_type=jnp.float32) + Ub_ref[...]
    i = _sigmoid(iuo[:, :DOUT] + wx[:, DOUT:2 * DOUT])
    u = jnp.tanh(iuo[:, DOUT:2 * DOUT] + wx[:, 2 * DOUT:3 * DOUT])
    o = _sigmoid(iuo[:, 2 * DOUT:] + wx[:, 3 * DOUT:])
    nc = i * u + bf
    c_ref[...] = nc
    h_ref[...] = o * jnp.tanh(nc)


def _full(shape):
    return pl.BlockSpec(shape, lambda j: (0, 0))


def _lvl0_call(x, Ww, Wb2, Ub2):
    return pl.pallas_call(
        _lvl0_body,
        grid=(NBLK,),
        in_specs=[
            pl.BlockSpec((NB, DIN), lambda j: (j, 0)),
            _full((DIN, 4 * DOUT)),
            _full((1, 4 * DOUT)),
            _full((1, 3 * DOUT)),
        ],
        out_specs=[pl.BlockSpec((NB, DOUT), lambda j: (j, 0))] * 2,
        out_shape=[jax.ShapeDtypeStruct((NP, DOUT), jnp.float32)] * 2,
    )(x, Ww, Wb2, Ub2)


def _lvl_call(g, x, Ww, Wb2, Ufw, Ufb2, Uw, Ub2):
    return pl.pallas_call(
        _lvl_body,
        grid=(NBLK,),
        in_specs=[
            pl.BlockSpec((NB, K * DHC), lambda j: (j, 0)),
            pl.BlockSpec((NB, DIN), lambda j: (j, 0)),
            _full((DIN, 4 * DOUT)),
            _full((1, 4 * DOUT)),
            _full((DOUT, DOUT)),
            _full((1, DOUT)),
            _full((DOUT, 3 * DOUT)),
            _full((1, 3 * DOUT)),
        ],
        out_specs=[pl.BlockSpec((NB, DOUT), lambda j: (j, 0))] * 2,
        out_shape=[jax.ShapeDtypeStruct((NP, DOUT), jnp.float32)] * 2,
    )(g, x, Ww, Wb2, Ufw, Ufb2, Uw, Ub2)


def kernel(tensor, indices, Uf_w, Uf_b, Uiuo_w, Uiuo_b, W_w, W_b):
    xpad = jnp.pad(tensor, ((0, 0), (0, NP - N), (0, 0)))
    Wb2 = W_b.reshape(1, 4 * DOUT)
    Ufb2 = Uf_b.reshape(1, DOUT)
    Ub2 = Uiuo_b.reshape(1, 3 * DOUT)

    h_prev, c_prev = _lvl0_call(xpad[0], W_w, Wb2, Ub2)
    res_h, res_c = [h_prev[:N]], [c_prev[:N]]
    for l in range(1, L):
        table = jnp.concatenate(
            [jnp.zeros((1, DHC), jnp.float32),
             jnp.concatenate([h_prev[:N], c_prev[:N]], axis=1)], axis=0)
        idx = jnp.maximum(indices[l], 0).reshape(-1)
        idx = jnp.pad(idx, (0, BG - N * K)).reshape(NW, NCH, CH)
        g = _gather_call()(table, idx)
        g = g[:NP * K].reshape(NP, K * DHC)
        h_prev, c_prev = _lvl_call(g, xpad[l], W_w, Wb2, Uf_w, Ufb2,
                                   Uiuo_w, Ub2)
        res_h.append(h_prev[:N])
        res_c.append(c_prev[:N])
    return jnp.stack(res_h), jnp.stack(res_c)
